# Initial kernel scaffold; baseline (speedup 1.0000x reference)
import jax
import jax.numpy as jnp
from jax.experimental import pallas as pl


def kernel(inputs, targets):
    raise NotImplementedError("write your pallas kernel here")


# speedup vs baseline: 1.6803x; 1.6803x over previous
import jax
import jax.numpy as jnp
from jax.experimental import pallas as pl

_NUM_CLASSES = 128
_ROWS_PER_BLOCK = 3200


def _class_sum(e):
    a = e[0:8]
    for i in range(1, 16):
        a = a + e[8 * i:8 * i + 8]
    b = a[0:4] + a[4:8]
    c = b[0:2] + b[2:4]
    return c[0:1] + c[1:2]


def _ce_block(x_ref, t_ref, o_ref):
    x = x_ref[...]
    t = t_ref[...]
    xt = jnp.transpose(x)
    x_max = jnp.max(xt, axis=0, keepdims=True)
    shifted = xt - x_max
    lse = jnp.log(_class_sum(jnp.exp(shifted)))
    cls = jax.lax.broadcasted_iota(jnp.int32, xt.shape, 0)
    shifted_t = jnp.sum(jnp.where(cls == t, shifted, 0.0), axis=0,
                        keepdims=True)
    o_ref[...] = -(shifted_t - lse)


def kernel(inputs, targets):
    n = targets.shape[0]
    r = _ROWS_PER_BLOCK
    t2 = targets.astype(jnp.int32).reshape(1, n)
    ps = pl.pallas_call(
        _ce_block,
        grid=(n // r,),
        in_specs=[
            pl.BlockSpec((r, _NUM_CLASSES), lambda i: (i, 0)),
            pl.BlockSpec((1, r), lambda i: (0, i)),
        ],
        out_specs=pl.BlockSpec((1, r), lambda i: (0, i)),
        out_shape=jax.ShapeDtypeStruct((1, n), jnp.float32),
    )(inputs, t2)
    per_sample = ps.reshape(n)
    per_class_sum = jax.ops.segment_sum(per_sample, targets,
                                        num_segments=_NUM_CLASSES)
    incorrect_loss = jnp.sum(per_class_sum) / n
    correct_loss = jnp.mean(per_sample)
    return jnp.abs(incorrect_loss - correct_loss)


# speedup vs baseline: 1.8522x; 1.1023x over previous
import jax
import jax.numpy as jnp
from jax.experimental import pallas as pl

_NUM_CLASSES = 128
_ROWS_PER_BLOCK = 12800


def _class_sum(e):
    a = e[0:8]
    for i in range(1, 16):
        a = a + e[8 * i:8 * i + 8]
    b = a[0:4] + a[4:8]
    c = b[0:2] + b[2:4]
    return c[0:1] + c[1:2]


def _ce_block(x_ref, t_ref, o_ref):
    x = x_ref[...]
    t = t_ref[...]
    xt = jnp.transpose(x)
    x_max = jnp.max(xt, axis=0, keepdims=True)
    shifted = xt - x_max
    lse = jnp.log(_class_sum(jnp.exp(shifted)))
    cls = jax.lax.broadcasted_iota(jnp.int32, xt.shape, 0)
    xt_t = jnp.sum(jnp.where(cls == t, xt, 0.0), axis=0, keepdims=True)
    shifted_t = xt_t - x_max
    o_ref[...] = -(shifted_t - lse)


def kernel(inputs, targets):
    n = targets.shape[0]
    r = _ROWS_PER_BLOCK
    t2 = targets.astype(jnp.int32).reshape(1, n)
    ps = pl.pallas_call(
        _ce_block,
        grid=(n // r,),
        in_specs=[
            pl.BlockSpec((r, _NUM_CLASSES), lambda i: (i, 0)),
            pl.BlockSpec((1, r), lambda i: (0, i)),
        ],
        out_specs=pl.BlockSpec((1, r), lambda i: (0, i)),
        out_shape=jax.ShapeDtypeStruct((1, n), jnp.float32),
    )(inputs, t2)
    per_sample = ps.reshape(n)
    per_class_sum = jax.ops.segment_sum(per_sample, targets,
                                        num_segments=_NUM_CLASSES)
    incorrect_loss = jnp.sum(per_class_sum) / n
    correct_loss = jnp.mean(per_sample)
    return jnp.abs(incorrect_loss - correct_loss)
